# baseline (speedup 1.0000x reference)
import functools

import jax
import jax.numpy as jnp
from jax import lax
from jax.experimental import pallas as pl
from jax.experimental.pallas import tpu as pltpu
from jax.experimental.pallas import tpu_sc as plsc

_B = 16384
_D = 3
_V = 1000000
_L = 16

_info = plsc.get_sparse_core_info()
_NC, _NS = _info.num_cores, _info.num_subcores
_NW = _NC * _NS
_BPW = _B // _NW
_NSEM = 8

_mesh = plsc.VectorSubcoreMesh(core_axis_name="c", subcore_axis_name="s")


@functools.partial(
    pl.kernel,
    mesh=_mesh,
    out_type=jax.ShapeDtypeStruct((_B, _D), jnp.float32),
    scratch_types=[
        pltpu.VMEM((_BPW,), jnp.int32),
        pltpu.VMEM((_BPW, _D), jnp.float32),
        [pltpu.SemaphoreType.DMA for _ in range(_NSEM)],
    ],
)
def _gather_kernel(idx_hbm, table_hbm, out_hbm, idx_v, rows_v, sems):
    wid = lax.axis_index("s") * _NC + lax.axis_index("c")
    pltpu.sync_copy(idx_hbm.at[pl.ds(wid * _BPW, _BPW)], idx_v)

    for g in range(_BPW // _L):
        v = idx_v[pl.ds(_L * g, _L)]
        for lane in range(_L):
            k = _L * g + lane
            pltpu.async_copy(
                table_hbm.at[v[lane]], rows_v.at[k], sems[k % _NSEM]
            )
    per_sem = _BPW // _NSEM
    for j in range(_NSEM):
        pltpu.make_async_copy(
            table_hbm.at[pl.ds(0, per_sem)],
            rows_v.at[pl.ds(j * per_sem, per_sem)],
            sems[j],
        ).wait()

    pltpu.sync_copy(rows_v, out_hbm.at[pl.ds(wid * _BPW, _BPW)])


def kernel(points, delta_translation):
    return _gather_kernel(points, delta_translation)


# speedup vs baseline: 1.0008x; 1.0008x over previous
import functools

import jax
import jax.numpy as jnp
from jax import lax
from jax.experimental import pallas as pl
from jax.experimental.pallas import tpu as pltpu
from jax.experimental.pallas import tpu_sc as plsc

_B = 16384
_D = 3
_V = 1000000
_L = 16

_info = plsc.get_sparse_core_info()
_NC, _NS = _info.num_cores, _info.num_subcores
_NW = _NC * _NS
_BPW = _B // _NW

_mesh = plsc.VectorSubcoreMesh(core_axis_name="c", subcore_axis_name="s")


@functools.partial(
    pl.kernel,
    mesh=_mesh,
    out_type=jax.ShapeDtypeStruct((_B, _D), jnp.float32),
    scratch_types=[
        pltpu.VMEM((_BPW,), jnp.int32),
        pltpu.VMEM((_BPW, _D), jnp.float32),
        pltpu.SemaphoreType.DMA,
    ],
)
def _gather_kernel(idx_hbm, table_hbm, out_hbm, idx_v, rows_v, sem):
    wid = lax.axis_index("s") * _NC + lax.axis_index("c")
    pltpu.sync_copy(idx_hbm.at[pl.ds(wid * _BPW, _BPW)], idx_v)

    for g in range(_BPW // _L):
        v = idx_v[pl.ds(_L * g, _L)]
        for lane in range(_L):
            pltpu.async_copy(
                table_hbm.at[v[lane]], rows_v.at[_L * g + lane], sem
            )
    pltpu.make_async_copy(
        table_hbm.at[pl.ds(0, _BPW)], rows_v, sem
    ).wait()

    pltpu.sync_copy(rows_v, out_hbm.at[pl.ds(wid * _BPW, _BPW)])


def kernel(points, delta_translation):
    return _gather_kernel(points, delta_translation)
